# baseline (speedup 1.0000x reference)
import functools

import jax
import jax.numpy as jnp
from jax import lax
from jax.experimental import pallas as pl
from jax.experimental.pallas import tpu as pltpu
from jax.experimental.pallas import tpu_sc as plsc

NC, NS = 2, 16
NW = NC * NS
LN_EPS_ = 1e-3


def _sc_gather(table, idx_flat):
    n_tok = idx_flat.shape[0]
    h = table.shape[1]
    b_per_w = n_tok // NW
    chunk = 64
    n_chunks = b_per_w // chunk

    mesh = plsc.VectorSubcoreMesh(
        core_axis_name="c", subcore_axis_name="s",
        num_cores=NC, num_subcores=NS)

    @functools.partial(
        pl.kernel,
        mesh=mesh,
        out_type=jax.ShapeDtypeStruct((n_tok, h), jnp.float32),
        scratch_types=[
            pltpu.VMEM((b_per_w,), jnp.int32),
            pltpu.VMEM((chunk, h), jnp.float32),
            pltpu.VMEM((chunk, h), jnp.float32),
            pltpu.SemaphoreType.DMA,
            pltpu.SemaphoreType.DMA,
        ],
    )
    def gather_kernel(table_hbm, idx_hbm, out_hbm, idx_v, rows0, rows1,
                      sem0, sem1):
        wid = lax.axis_index("s") * NC + lax.axis_index("c")
        base = wid * b_per_w
        bufs = (rows0, rows1)
        sems = (sem0, sem1)

        pltpu.sync_copy(idx_hbm.at[pl.ds(base, b_per_w)], idx_v)

        def start(c):
            return pltpu.async_copy(
                table_hbm.at[idx_v.at[pl.ds(c * chunk, chunk)]],
                bufs[c % 2], sems[c % 2])

        cp = start(0)
        for c in range(n_chunks):
            cp.wait()
            if c + 1 < n_chunks:
                cp = start(c + 1)
            pltpu.sync_copy(bufs[c % 2],
                            out_hbm.at[pl.ds(base + c * chunk, chunk)])

    return gather_kernel(table, idx_flat)


def _tc_add_ln(gathered, ttf, type_emb, pos_emb, gamma, beta,
               n_tok_total, blk_off, out_prev):
    csize, h = gathered.shape
    seq = pos_emb.shape[0]
    t_blk = 1024
    grid = csize // t_blk
    pos_blk = seq // t_blk if seq >= t_blk else 1

    def body(x_ref, tt_ref, te_ref, pos_ref, g_ref, b_ref, *rest):
        o_ref = rest[-1]
        i = pl.program_id(0) + blk_off
        x = x_ref[...]
        t0 = te_ref[0:1, :]
        dt = te_ref[1:2, :] - t0
        pos = pos_ref[pl.ds((i % pos_blk) * t_blk, t_blk), :]
        x = x + pos + t0 + tt_ref[...] * dt
        m = jnp.mean(x, axis=-1, keepdims=True)
        d = x - m
        v = jnp.mean(d * d, axis=-1, keepdims=True)
        o_ref[...] = d * lax.rsqrt(v + LN_EPS_) * g_ref[...] + b_ref[...]

    in_specs = [
        pl.BlockSpec((t_blk, h), lambda i: (i, 0)),
        pl.BlockSpec((t_blk, 1), lambda i: (i, 0)),
        pl.BlockSpec((2, h), lambda i: (0, 0)),
        pl.BlockSpec((seq, h), lambda i: (0, 0)),
        pl.BlockSpec((1, h), lambda i: (0, 0)),
        pl.BlockSpec((1, h), lambda i: (0, 0)),
    ]
    args = [gathered, ttf, type_emb, pos_emb, gamma, beta]
    aliases = {}
    if out_prev is not None:
        in_specs.append(pl.BlockSpec((8, 128), lambda i: (0, 0)))
        args.append(out_prev)
        aliases = {6: 0}

    return pl.pallas_call(
        body,
        grid=(grid,),
        in_specs=in_specs,
        out_specs=pl.BlockSpec((t_blk, h), lambda i: (i + blk_off, 0)),
        out_shape=jax.ShapeDtypeStruct((n_tok_total, h), jnp.float32),
        input_output_aliases=aliases,
    )(*args)


def kernel(input_ids, token_type_ids, word_embeddings, token_type_embeddings,
           position_embeddings, ln_gamma, ln_beta):
    b, s = input_ids.shape
    h = word_embeddings.shape[1]
    n_tok = b * s
    idx_flat = input_ids.reshape(-1).astype(jnp.int32)
    ttf = token_type_ids.reshape(-1, 1).astype(jnp.float32)
    pos = lax.dynamic_slice_in_dim(position_embeddings, 0, s, axis=0)
    gamma = ln_gamma.reshape(1, h)
    beta = ln_beta.reshape(1, h)

    n_split = 2
    csize = n_tok // n_split
    blk_per_chunk = csize // 1024
    out = None
    for k in range(n_split):
        g_k = _sc_gather(word_embeddings,
                         lax.dynamic_slice_in_dim(idx_flat, k * csize, csize))
        tt_k = lax.dynamic_slice_in_dim(ttf, k * csize, csize)
        out = _tc_add_ln(g_k, tt_k, token_type_embeddings, pos, gamma, beta,
                         n_tok_total=n_tok, blk_off=k * blk_per_chunk,
                         out_prev=out)
    return out.reshape(b, s, h)


# speedup vs baseline: 1.0390x; 1.0390x over previous
import functools

import jax
import jax.numpy as jnp
from jax import lax
from jax.experimental import pallas as pl
from jax.experimental.pallas import tpu as pltpu
from jax.experimental.pallas import tpu_sc as plsc

NC, NS = 2, 16
NW = NC * NS
LN_EPS_ = 1e-3
N_SPLIT = 2
T_BLK = 1024


def _sc_gather_chunk(table, idx_flat, batch, seq, k):
    h = table.shape[1]
    s_chunk = seq // N_SPLIT
    csize = batch * s_chunk
    b_per_w = csize // NW
    w_per_batch = NW // batch
    chunk = min(64, b_per_w)
    n_chunks = b_per_w // chunk

    mesh = plsc.VectorSubcoreMesh(
        core_axis_name="c", subcore_axis_name="s",
        num_cores=NC, num_subcores=NS)

    @functools.partial(
        pl.kernel,
        mesh=mesh,
        out_type=jax.ShapeDtypeStruct((csize, h), jnp.float32),
        scratch_types=[
            pltpu.VMEM((b_per_w,), jnp.int32),
            pltpu.VMEM((chunk, h), jnp.float32),
            pltpu.VMEM((chunk, h), jnp.float32),
            pltpu.SemaphoreType.DMA,
            pltpu.SemaphoreType.DMA,
        ],
    )
    def gather_kernel(table_hbm, idx_hbm, out_hbm, idx_v, rows0, rows1,
                      sem0, sem1):
        wid = lax.axis_index("s") * NC + lax.axis_index("c")
        src_base = ((wid // w_per_batch) * seq + k * s_chunk
                    + (wid % w_per_batch) * b_per_w)
        dst_base = wid * b_per_w
        bufs = (rows0, rows1)
        sems = (sem0, sem1)

        pltpu.sync_copy(idx_hbm.at[pl.ds(src_base, b_per_w)], idx_v)

        def start(c):
            return pltpu.async_copy(
                table_hbm.at[idx_v.at[pl.ds(c * chunk, chunk)]],
                bufs[c % 2], sems[c % 2])

        cp = start(0)
        for c in range(n_chunks):
            cp.wait()
            if c + 1 < n_chunks:
                cp = start(c + 1)
            pltpu.sync_copy(bufs[c % 2],
                            out_hbm.at[pl.ds(dst_base + c * chunk, chunk)])

    return gather_kernel(table, idx_flat)


def _tc_add_ln_chunk(gathered, ttf, type_emb, pos_emb, gamma, beta,
                     batch, seq, k, out_prev):
    csize, h = gathered.shape
    s_chunk = csize // batch
    grid = csize // T_BLK
    blk_per_s_chunk = s_chunk // T_BLK
    blk_per_seq = seq // T_BLK

    def body(x_ref, tt_ref, te_ref, pos_ref, g_ref, b_ref, *rest):
        o_ref = rest[-1]
        x = x_ref[...]
        t0 = te_ref[0:1, :]
        dt = te_ref[1:2, :] - t0
        x = x + pos_ref[...] + t0 + tt_ref[...] * dt
        m = jnp.mean(x, axis=-1, keepdims=True)
        d = x - m
        v = jnp.mean(d * d, axis=-1, keepdims=True)
        o_ref[...] = d * lax.rsqrt(v + LN_EPS_) * g_ref[...] + b_ref[...]

    def tok_map(i):
        return ((i // blk_per_s_chunk) * blk_per_seq
                + k * blk_per_s_chunk + i % blk_per_s_chunk, 0)

    in_specs = [
        pl.BlockSpec((T_BLK, h), lambda i: (i, 0)),
        pl.BlockSpec((T_BLK, 1), tok_map),
        pl.BlockSpec((2, h), lambda i: (0, 0)),
        pl.BlockSpec((T_BLK, h),
                     lambda i: (k * blk_per_s_chunk + i % blk_per_s_chunk, 0)),
        pl.BlockSpec((1, h), lambda i: (0, 0)),
        pl.BlockSpec((1, h), lambda i: (0, 0)),
    ]
    args = [gathered, ttf, type_emb, pos_emb, gamma, beta]
    aliases = {}
    if out_prev is not None:
        in_specs.append(pl.BlockSpec((8, 128), lambda i: (0, 0)))
        args.append(out_prev)
        aliases = {6: 0}

    return pl.pallas_call(
        body,
        grid=(grid,),
        in_specs=in_specs,
        out_specs=pl.BlockSpec((T_BLK, h), tok_map),
        out_shape=jax.ShapeDtypeStruct((batch * seq, h), jnp.float32),
        input_output_aliases=aliases,
    )(*args)


def kernel(input_ids, token_type_ids, word_embeddings, token_type_embeddings,
           position_embeddings, ln_gamma, ln_beta):
    b, s = input_ids.shape
    h = word_embeddings.shape[1]
    idx_flat = input_ids.reshape(-1).astype(jnp.int32)
    ttf = token_type_ids.reshape(-1, 1).astype(jnp.float32)
    if position_embeddings.shape[0] != s:
        pos = lax.dynamic_slice_in_dim(position_embeddings, 0, s, axis=0)
    else:
        pos = position_embeddings
    gamma = ln_gamma.reshape(1, h)
    beta = ln_beta.reshape(1, h)

    out = None
    for k in range(N_SPLIT):
        g_k = _sc_gather_chunk(word_embeddings, idx_flat, b, s, k)
        out = _tc_add_ln_chunk(g_k, ttf, token_type_embeddings, pos,
                               gamma, beta, b, s, k, out)
    return out.reshape(b, s, h)
